# Initial kernel scaffold; baseline (speedup 1.0000x reference)
import jax
import jax.numpy as jnp
from jax.experimental import pallas as pl


def kernel(x, edge_index, edge_attr, We1, be1, We2, be2, W1, b1, Wm, bm, Wo, bo, Wp1, bp1, Wp2, bp2, Wp3, bp3):
    raise NotImplementedError("write your pallas kernel here")


# speedup vs baseline: 7.4946x; 7.4946x over previous
import functools

import jax
import jax.numpy as jnp
from jax import lax
from jax.experimental import pallas as pl
from jax.experimental.pallas import tpu as pltpu
from jax.experimental.pallas import tpu_sc as plsc

N = 10000
NP = 10112
E = 320000
H = 64
NC, NS = 2, 16
NW = NC * NS
EPW = 10240
NCH = 80
CB = 128
EP = NW * EPW
RPS = NP // NS
F32 = jnp.float32


def _edge_mlp_body(ea_ref, we1_ref, be1_ref, we2_ref, be2_ref, out_ref):
    h = jnp.dot(ea_ref[...], we1_ref[...], preferred_element_type=F32)
    h = jnp.maximum(h + be1_ref[...], 0.0)
    ef = jnp.dot(h, we2_ref[...], preferred_element_type=F32) + be2_ref[...]
    out_ref[...] = jnp.concatenate(
        [ef, jnp.zeros((ef.shape[0], 128 - H), F32)], axis=1)


_EDGE_BLK = 2048
_edge_mlp = pl.pallas_call(
    _edge_mlp_body,
    grid=(EP // _EDGE_BLK,),
    in_specs=[
        pl.BlockSpec((_EDGE_BLK, 4), lambda i: (i, 0)),
        pl.BlockSpec((4, H), lambda i: (0, 0)),
        pl.BlockSpec((1, H), lambda i: (0, 0)),
        pl.BlockSpec((H, H), lambda i: (0, 0)),
        pl.BlockSpec((1, H), lambda i: (0, 0)),
    ],
    out_specs=pl.BlockSpec((_EDGE_BLK, 128), lambda i: (i, 0)),
    out_shape=jax.ShapeDtypeStruct((EP, 128), F32),
)


def _layer0_body(x_ref, nef_ref, cs_ref, cd_ref, w1_ref, b1_ref,
                 t_ref, u_ref, dinv_ref):
    cs = cs_ref[0] + cs_ref[1]
    cd = cd_ref[0] + cd_ref[1]
    deg = jnp.maximum(cs[:, 0:1] + cd[:, 0:1], 1.0)
    dinv = lax.rsqrt(cd[:, 0:1] + 1.0)
    nefn = (nef_ref[0, :, 0:H] + nef_ref[1, :, 0:H]) * (0.5 / deg)
    xv = x_ref[...]
    hw = jnp.dot(xv, w1_ref[...], preferred_element_type=F32)
    hw = hw + jnp.dot(nefn, w1_ref[0:H, :], preferred_element_type=F32)
    t_ref[...] = jnp.concatenate(
        [dinv * hw, jnp.zeros((hw.shape[0], 128 - H), F32)], axis=1)
    u_ref[...] = (dinv * dinv) * hw + b1_ref[...]
    dinv_ref[...] = dinv


_L0B = NP // 8
_layer0 = pl.pallas_call(
    _layer0_body,
    grid=(8,),
    in_specs=[
        pl.BlockSpec((_L0B, 128), lambda i: (i, 0)),
        pl.BlockSpec((NC, _L0B, 128), lambda i: (0, i, 0)),
        pl.BlockSpec((NC, _L0B, 128), lambda i: (0, i, 0)),
        pl.BlockSpec((NC, _L0B, 128), lambda i: (0, i, 0)),
        pl.BlockSpec((128, H), lambda i: (0, 0)),
        pl.BlockSpec((1, H), lambda i: (0, 0)),
    ],
    out_specs=(
        pl.BlockSpec((_L0B, 128), lambda i: (i, 0)),
        pl.BlockSpec((_L0B, H), lambda i: (i, 0)),
        pl.BlockSpec((_L0B, 1), lambda i: (i, 0)),
    ),
    out_shape=(
        jax.ShapeDtypeStruct((NP, 128), F32),
        jax.ShapeDtypeStruct((NP, H), F32),
        jax.ShapeDtypeStruct((NP, 1), F32),
    ),
)


def _layer_body(agg_ref, u_ref, dinv_ref, w_ref, b_ref, t_ref, uo_ref):
    dinv = dinv_ref[...]
    aggs = agg_ref[0, :, 0:H] + agg_ref[1, :, 0:H]
    h = jnp.maximum(dinv * aggs + u_ref[...], 0.0)
    hw = jnp.dot(h, w_ref[...], preferred_element_type=F32)
    t_ref[...] = jnp.concatenate(
        [dinv * hw, jnp.zeros((hw.shape[0], 128 - H), F32)], axis=1)
    uo_ref[...] = (dinv * dinv) * hw + b_ref[...]


_layer = pl.pallas_call(
    _layer_body,
    grid=(8,),
    in_specs=[
        pl.BlockSpec((NC, _L0B, 128), lambda i: (0, i, 0)),
        pl.BlockSpec((_L0B, H), lambda i: (i, 0)),
        pl.BlockSpec((_L0B, 1), lambda i: (i, 0)),
        pl.BlockSpec((H, H), lambda i: (0, 0)),
        pl.BlockSpec((1, H), lambda i: (0, 0)),
    ],
    out_specs=(
        pl.BlockSpec((_L0B, 128), lambda i: (i, 0)),
        pl.BlockSpec((_L0B, H), lambda i: (i, 0)),
    ),
    out_shape=(
        jax.ShapeDtypeStruct((NP, 128), F32),
        jax.ShapeDtypeStruct((NP, H), F32),
    ),
)


def _head_body(agg_ref, u_ref, dinv_ref, wp1_ref, bp1_ref, wp2_ref, bp2_ref,
               wp3_ref, bp3_ref, out_ref):
    h = dinv_ref[...] * (agg_ref[0, :, 0:H] + agg_ref[1, :, 0:H]) + u_ref[...]
    mask = (lax.broadcasted_iota(jnp.int32, (NP, 1), 0) < N).astype(F32)
    hm = h * mask
    m = jnp.sum(hm, axis=0, keepdims=True) * (1.0 / N)
    d = (h - m) * mask
    sd = jnp.sqrt(jnp.sum(d * d, axis=0, keepdims=True) * (1.0 / (N - 1)))
    comb = jnp.concatenate([m, m, sd], axis=1)
    z = jnp.maximum(jnp.dot(comb, wp1_ref[...], preferred_element_type=F32)
                    + bp1_ref[...], 0.0)
    z = jnp.maximum(jnp.dot(z, wp2_ref[...], preferred_element_type=F32)
                    + bp2_ref[...], 0.0)
    logit = jnp.dot(z, wp3_ref[...], preferred_element_type=F32) + bp3_ref[...]
    out_ref[...] = 1.0 / (1.0 + jnp.exp(-logit))


_head = pl.pallas_call(
    _head_body,
    out_shape=jax.ShapeDtypeStruct((1, 1), F32),
)


_MESH = plsc.VectorSubcoreMesh(core_axis_name="c", subcore_axis_name="s")


@functools.partial(
    pl.kernel,
    out_type=jax.ShapeDtypeStruct((NC, NP, 128), F32),
    mesh=_MESH,
    scratch_types=[
        pltpu.VMEM((NCH, CB), jnp.int32),
        pltpu.VMEM((NCH, CB), jnp.int32),
        pltpu.VMEM((CB, 128), F32),
        pltpu.VMEM_SHARED((NP, 128), F32),
    ],
)
def _edge_scatter(ef, srcs, dsts, z128, nef_out, idx_s, idx_d, efbuf, acc_nef):
    c = lax.axis_index("c")
    s = lax.axis_index("s")
    w = c * NS + s
    r0 = s * RPS
    pltpu.sync_copy(z128.at[pl.ds(r0, RPS)], acc_nef.at[pl.ds(r0, RPS)])
    pltpu.sync_copy(srcs.at[w], idx_s)
    pltpu.sync_copy(dsts.at[w], idx_d)
    plsc.subcore_barrier()

    @pl.loop(0, NCH)
    def _edge_loop(j):
        pltpu.sync_copy(ef.at[pl.ds(w * EPW + j * CB, CB)], efbuf)
        pltpu.sync_copy(efbuf, acc_nef.at[idx_s.at[j]], add=True)
        pltpu.sync_copy(efbuf, acc_nef.at[idx_d.at[j]], add=True)
    plsc.subcore_barrier()
    pltpu.sync_copy(acc_nef.at[pl.ds(r0, RPS)], nef_out.at[c, pl.ds(r0, RPS)])


@functools.partial(
    pl.kernel,
    out_type=jax.ShapeDtypeStruct((NC, NP, 128), F32),
    mesh=_MESH,
    scratch_types=[
        pltpu.VMEM((NCH, CB), jnp.int32),
        pltpu.VMEM((CB, 128), F32),
        pltpu.VMEM_SHARED((NP, 128), F32),
    ],
)
def _count_scatter(idxs, z128, ones, cnt_out, idx_v, onesbuf, acc):
    c = lax.axis_index("c")
    s = lax.axis_index("s")
    w = c * NS + s
    r0 = s * RPS
    pltpu.sync_copy(z128.at[pl.ds(r0, RPS)], acc.at[pl.ds(r0, RPS)])
    pltpu.sync_copy(idxs.at[w], idx_v)
    plsc.subcore_barrier()

    @pl.loop(0, NCH)
    def _count_loop(j):
        pltpu.sync_copy(ones, onesbuf)
        pltpu.sync_copy(onesbuf, acc.at[idx_v.at[j]], add=True)
    plsc.subcore_barrier()
    pltpu.sync_copy(acc.at[pl.ds(r0, RPS)], cnt_out.at[c, pl.ds(r0, RPS)])


@functools.partial(
    pl.kernel,
    out_type=jax.ShapeDtypeStruct((NC, NP, 128), F32),
    mesh=_MESH,
    scratch_types=[
        pltpu.VMEM((NCH, CB), jnp.int32),
        pltpu.VMEM((NCH, CB), jnp.int32),
        pltpu.VMEM((CB, 128), F32),
        pltpu.VMEM_SHARED((NP, 128), F32),
        pltpu.SemaphoreType.DMA,
    ],
)
def _gcn_agg(table, srcs, dsts, z128, agg_out, idx_s, idx_d, rowbuf, acc, sem):
    c = lax.axis_index("c")
    s = lax.axis_index("s")
    w = c * NS + s
    r0 = s * RPS
    pltpu.sync_copy(z128.at[pl.ds(r0, RPS)], acc.at[pl.ds(r0, RPS)])
    pltpu.sync_copy(srcs.at[w], idx_s)
    pltpu.sync_copy(dsts.at[w], idx_d)
    plsc.subcore_barrier()

    @pl.loop(0, NCH)
    def _agg_loop(j):
        pltpu.async_copy(table.at[idx_s.at[j]], rowbuf, sem).wait()
        pltpu.sync_copy(rowbuf, acc.at[idx_d.at[j]], add=True)
    plsc.subcore_barrier()
    pltpu.sync_copy(acc.at[pl.ds(r0, RPS)], agg_out.at[c, pl.ds(r0, RPS)])


def kernel(x, edge_index, edge_attr, We1, be1, We2, be2, W1, b1, Wm, bm,
           Wo, bo, Wp1, bp1, Wp2, bp2, Wp3, bp3):
    src = edge_index[0]
    dst = edge_index[1]
    npad = EPW - E // NW
    pad_idx = (N + (jnp.arange(npad, dtype=jnp.int32) % (NP - N)))
    padw = jnp.broadcast_to(pad_idx, (NW, npad))
    srcs = jnp.concatenate([src.reshape(NW, E // NW), padw], axis=1)
    dsts = jnp.concatenate([dst.reshape(NW, E // NW), padw], axis=1)
    srcs = srcs.reshape(NW, NCH, CB)
    dsts = dsts.reshape(NW, NCH, CB)
    ea_p = jnp.concatenate(
        [edge_attr.reshape(NW, E // NW, 4), jnp.zeros((NW, npad, 4), F32)],
        axis=1).reshape(EP, 4)
    x_p = jnp.concatenate([x, jnp.zeros((NP - N, x.shape[1]), F32)], axis=0)
    z128 = jnp.zeros((NP, 128), F32)
    on128 = jnp.ones((CB, 128), F32)

    ef = _edge_mlp(ea_p, We1, be1.reshape(1, H), We2, be2.reshape(1, H))
    nef2 = _edge_scatter(ef, srcs, dsts, z128)
    cs2 = _count_scatter(srcs, z128, on128)
    cd2 = _count_scatter(dsts, z128, on128)
    t, u, dinv = _layer0(x_p, nef2, cs2, cd2, W1, b1.reshape(1, H))
    agg = _gcn_agg(t, srcs, dsts, z128)
    t, u = _layer(agg, u, dinv, Wm, bm.reshape(1, H))
    agg = _gcn_agg(t, srcs, dsts, z128)
    t, u = _layer(agg, u, dinv, Wo, bo.reshape(1, H))
    agg = _gcn_agg(t, srcs, dsts, z128)
    out = _head(agg, u, dinv, Wp1, bp1.reshape(1, H), Wp2, bp2.reshape(1, 32),
                Wp3, bp3.reshape(1, 1))
    return out.reshape(1)
